# baseline (speedup 1.0000x reference)
import functools

import jax
import jax.numpy as jnp
from jax import lax
from jax.experimental import pallas as pl
from jax.experimental.pallas import tpu as pltpu
from jax.experimental.pallas import tpu_sc as plsc

NUM_CORES = 2
NUM_SUBCORES = 16
NUM_WORKERS = NUM_CORES * NUM_SUBCORES

_SC_PARAMS = pltpu.CompilerParams(use_tc_tiling_on_sc=False)


def _pick_batch(edges_per_worker):
    for cand in range(128, 0, -1):
        if edges_per_worker % cand == 0:
            return cand
    return 1


def _row_chunks(n):
    for nz in range(NUM_SUBCORES, 0, -1):
        if n % nz == 0 and (n // nz) % 8 == 0:
            return nz, n // nz
    return 1, n


def _sc_degree(src2d, dst2d, zeros16, ones_rows, n):
    nbt, b = src2d.shape
    nbw = nbt // NUM_WORKERS
    nz, zch = _row_chunks(n)
    mesh = plsc.VectorSubcoreMesh(core_axis_name="c", subcore_axis_name="s")

    @functools.partial(
        pl.kernel,
        out_type=jax.ShapeDtypeStruct((2, NUM_CORES, n, 16), jnp.float32),
        mesh=mesh,
        scratch_types=[
            pltpu.VMEM((nbw, b), jnp.int32),
            pltpu.VMEM((nbw, b), jnp.int32),
            pltpu.VMEM((b, 16), jnp.float32),
            pltpu.VMEM_SHARED((n, 16), jnp.float32),
            pltpu.VMEM_SHARED((n, 16), jnp.float32),
            pltpu.SemaphoreType.DMA,
        ],
        compiler_params=_SC_PARAMS,
    )
    def deg_kernel(src_hbm, dst_hbm, z_hbm, ones_hbm, out_hbm,
                   sidx, didx, ones_v, dsrc_sh, ddst_sh, sem):
        c = lax.axis_index("c")
        s = lax.axis_index("s")
        wid = c * NUM_SUBCORES + s
        r0 = s * zch
        pltpu.sync_copy(src_hbm.at[pl.ds(wid * nbw, nbw)], sidx)
        pltpu.sync_copy(dst_hbm.at[pl.ds(wid * nbw, nbw)], didx)
        pltpu.sync_copy(ones_hbm, ones_v)

        @pl.when(s < nz)
        def _():
            pltpu.sync_copy(z_hbm.at[pl.ds(r0, zch)],
                            dsrc_sh.at[pl.ds(r0, zch)])
            pltpu.sync_copy(z_hbm.at[pl.ds(r0, zch)],
                            ddst_sh.at[pl.ds(r0, zch)])

        plsc.subcore_barrier()

        @pl.loop(0, nbw)
        def _(i):
            pltpu.async_copy(ones_v, dsrc_sh.at[sidx.at[i]], sem, add=True)
            pltpu.async_copy(ones_v, ddst_sh.at[didx.at[i]], sem, add=True)

        @pl.loop(0, 2 * nbw)
        def _(i):
            pltpu.make_async_copy(ones_v, dsrc_sh.at[sidx.at[0]], sem).wait()

        plsc.subcore_barrier()

        @pl.when(s < nz)
        def _():
            pltpu.sync_copy(dsrc_sh.at[pl.ds(r0, zch)],
                            out_hbm.at[0, c, pl.ds(r0, zch)])
            pltpu.sync_copy(ddst_sh.at[pl.ds(r0, zch)],
                            out_hbm.at[1, c, pl.ds(r0, zch)])

    return deg_kernel(src2d, dst2d, zeros16, ones_rows)


def _sc_prop(u_or_partials, src2d, dst2d, zeros, n, cp, rio=None):
    fused = rio is not None
    nbt, b = src2d.shape
    nbw = nbt // NUM_WORKERS
    nz, zch = _row_chunks(n)
    mch = 200 if zch % 200 == 0 else zch
    ncnk = zch // mch
    mesh = plsc.VectorSubcoreMesh(core_axis_name="c", subcore_axis_name="s")

    out_type = [jax.ShapeDtypeStruct((NUM_CORES, n, cp), jnp.float32)]
    if fused:
        out_type.append(jax.ShapeDtypeStruct((NUM_CORES, n, cp), jnp.float32))

    combine_scratch = []
    if fused:
        combine_scratch = [
            pltpu.VMEM((mch, cp), jnp.float32),
            pltpu.VMEM((mch, cp), jnp.float32),
            pltpu.VMEM((mch, cp), jnp.float32),
            pltpu.VMEM((mch, cp), jnp.float32),
        ]

    @functools.partial(
        pl.kernel,
        out_type=out_type,
        mesh=mesh,
        scratch_types=combine_scratch + [
            pltpu.VMEM((nbw, b), jnp.int32),
            pltpu.VMEM((nbw, b), jnp.int32),
            pltpu.VMEM((b, cp), jnp.float32),
            pltpu.VMEM((b, cp), jnp.float32),
            pltpu.VMEM((b, cp), jnp.float32),
            pltpu.VMEM((b, cp), jnp.float32),
            pltpu.VMEM_SHARED((n, cp), jnp.float32),
            pltpu.SemaphoreType.DMA,
            pltpu.SemaphoreType.DMA,
            pltpu.SemaphoreType.DMA,
            pltpu.SemaphoreType.DMA,
            pltpu.SemaphoreType.DMA,
            pltpu.SemaphoreType.DMA,
            pltpu.SemaphoreType.DMA,
            pltpu.SemaphoreType.DMA,
        ],
        compiler_params=_SC_PARAMS,
    )
    def hop_kernel(*refs):
        if fused:
            (up_hbm, rio_hbm, src_hbm, dst_hbm, z_hbm, out_hbm, u_scr,
             pc0, pc1, rioc, ucnk,
             sidx, didx, r0b, r1b, r2b, r3b, acc_sh,
             g0, g1, g2, g3, s0, s1, s2, s3) = refs
        else:
            (up_hbm, src_hbm, dst_hbm, z_hbm, out_hbm,
             sidx, didx, r0b, r1b, r2b, r3b, acc_sh,
             g0, g1, g2, g3, s0, s1, s2, s3) = refs
        rows = (r0b, r1b, r2b, r3b)
        gsem = (g0, g1, g2, g3)
        ssem = (s0, s1, s2, s3)
        c = lax.axis_index("c")
        s = lax.axis_index("s")
        wid = c * NUM_SUBCORES + s
        r0 = s * zch
        pltpu.sync_copy(src_hbm.at[pl.ds(wid * nbw, nbw)], sidx)
        pltpu.sync_copy(dst_hbm.at[pl.ds(wid * nbw, nbw)], didx)

        @pl.when(s < nz)
        def _():
            pltpu.sync_copy(z_hbm.at[pl.ds(r0, zch)],
                            acc_sh.at[pl.ds(r0, zch)])

        if fused:
            @pl.when(s < nz)
            def _():
                for t in range(ncnk):
                    rr = r0 + t * mch
                    pltpu.sync_copy(up_hbm.at[0, pl.ds(rr, mch)], pc0)
                    pltpu.sync_copy(up_hbm.at[1, pl.ds(rr, mch)], pc1)
                    pltpu.sync_copy(rio_hbm.at[pl.ds(rr, mch)], rioc)

                    @pl.loop(0, mch, step=8)
                    def _(r):
                        for dr in range(8):
                            for q in range(cp // 16):
                                sl = (r + dr, pl.ds(q * 16, 16))
                                ucnk[sl] = rioc[sl] * (pc0[sl] + pc1[sl])

                    pltpu.sync_copy(ucnk, u_scr.at[c, pl.ds(rr, mch)])

        plsc.subcore_barrier()
        u_hbm = u_scr.at[c] if fused else up_hbm

        nv = nbw + 4

        @pl.loop(0, nv, step=4)
        def _(v):
            for j in range(4):
                i = v + j

                @pl.when((i >= 4) & (i < nbw))
                def _():
                    pltpu.make_async_copy(
                        rows[j], acc_sh.at[didx.at[0]], ssem[j]).wait()

                @pl.when(i < nbw)
                def _():
                    pltpu.async_copy(u_hbm.at[sidx.at[i]], rows[j], gsem[j])

                k = i - 2
                jb = (j + 2) % 4

                @pl.when((k >= 0) & (k < nbw))
                def _():
                    pltpu.make_async_copy(
                        u_hbm.at[sidx.at[0]], rows[jb], gsem[jb]).wait()
                    pltpu.async_copy(rows[jb], acc_sh.at[didx.at[k]],
                                     ssem[jb], add=True)

        for j in range(4):
            pltpu.make_async_copy(rows[j], acc_sh.at[didx.at[0]],
                                  ssem[j]).wait()

        plsc.subcore_barrier()

        @pl.when(s < nz)
        def _():
            pltpu.sync_copy(acc_sh.at[pl.ds(r0, zch)],
                            out_hbm.at[c, pl.ds(r0, zch)])

    if fused:
        return hop_kernel(u_or_partials, rio, src2d, dst2d, zeros)[0]
    return hop_kernel(u_or_partials, src2d, dst2d, zeros)[0]


def _tc_project(feature, w_padded, n, cp):
    def body(f_ref, w_ref, o_ref):
        o_ref[...] = jnp.dot(f_ref[...], w_ref[...],
                             preferred_element_type=jnp.float32)

    return pl.pallas_call(
        body, out_shape=jax.ShapeDtypeStruct((n, cp), jnp.float32),
    )(feature, w_padded)


def _tc_prep(y0, d_out0, d_out1, d_in0, d_in1, n, cp):
    def body(y_ref, do0, do1, di0, di1, u_ref, rio_ref, rin_ref):
        deg_out = do0[...] + do1[...]
        deg_in = di0[...] + di1[...]
        rout = lax.rsqrt(jnp.maximum(deg_out, 1.0))
        rin = lax.rsqrt(jnp.maximum(deg_in, 1.0))
        u_ref[...] = y_ref[...] * rout
        rio_ref[...] = jnp.broadcast_to(rin * rout, (n, cp))
        rin_ref[...] = rin

    return pl.pallas_call(
        body,
        out_shape=[
            jax.ShapeDtypeStruct((n, cp), jnp.float32),
            jax.ShapeDtypeStruct((n, cp), jnp.float32),
            jax.ShapeDtypeStruct((n, 1), jnp.float32),
        ],
    )(y0, d_out0, d_out1, d_in0, d_in1)


def _tc_final(p1, p2, p3, y0, rin, b2d, n, c_out):
    def body(p1_ref, p2_ref, p3_ref, y_ref, rin_ref, b_ref, o_ref):
        psum = ((p1_ref[0] + p1_ref[1]) + (p2_ref[0] + p2_ref[1])
                + (p3_ref[0] + p3_ref[1]))
        res = (y_ref[...] + psum * rin_ref[...]) * 0.25
        o_ref[...] = res[:, :c_out] + b_ref[...]

    return pl.pallas_call(
        body, out_shape=jax.ShapeDtypeStruct((n, c_out), jnp.float32),
    )(p1, p2, p3, y0, rin, b2d)


def kernel(feature, edge_index, W, b):
    n, d = feature.shape
    c_out = W.shape[1]
    e = edge_index.shape[1]
    cp = ((c_out + 15) // 16) * 16

    epw = e // NUM_WORKERS
    batch = _pick_batch(epw)
    nbt = e // batch

    src2d = edge_index[0].reshape(nbt, batch)
    dst2d = edge_index[1].reshape(nbt, batch)
    w_padded = jnp.pad(W, ((0, 0), (0, cp - c_out)))
    zeros16 = jnp.zeros((n, 16), jnp.float32)
    ones_rows = jnp.ones((batch, 16), jnp.float32)
    zeros_cp = jnp.zeros((n, cp), jnp.float32)
    b2d = b.reshape(1, c_out)

    y0 = _tc_project(feature, w_padded, n, cp)
    degs = _sc_degree(src2d, dst2d, zeros16, ones_rows, n)
    d_out0 = degs[0, 0, :, 0:1]
    d_out1 = degs[0, 1, :, 0:1]
    d_in0 = degs[1, 0, :, 0:1]
    d_in1 = degs[1, 1, :, 0:1]

    u0, rio, rin = _tc_prep(y0, d_out0, d_out1, d_in0, d_in1, n, cp)
    p1 = _sc_prop(u0, src2d, dst2d, zeros_cp, n, cp)
    p2 = _sc_prop(p1, src2d, dst2d, zeros_cp, n, cp, rio=rio)
    p3 = _sc_prop(p2, src2d, dst2d, zeros_cp, n, cp, rio=rio)
    return _tc_final(p1, p2, p3, y0, rin, b2d, n, c_out)


# speedup vs baseline: 1.0351x; 1.0351x over previous
import functools

import jax
import jax.numpy as jnp
from jax import lax
from jax.experimental import pallas as pl
from jax.experimental.pallas import tpu as pltpu
from jax.experimental.pallas import tpu_sc as plsc

NUM_CORES = 2
NUM_SUBCORES = 16
NUM_WORKERS = NUM_CORES * NUM_SUBCORES

_SC_PARAMS = pltpu.CompilerParams(use_tc_tiling_on_sc=False)


def _pick_batch(edges_per_worker):
    for cand in range(128, 0, -1):
        if edges_per_worker % cand == 0:
            return cand
    return 1


def _row_chunks(n):
    for nz in range(NUM_SUBCORES, 0, -1):
        if n % nz == 0 and (n // nz) % 8 == 0:
            return nz, n // nz
    return 1, n


def _sc_degree(src2d, dst2d, zeros16, ones_rows, n):
    nbt, b = src2d.shape
    nbw = nbt // NUM_WORKERS
    nz, zch = _row_chunks(n)
    mesh = plsc.VectorSubcoreMesh(core_axis_name="c", subcore_axis_name="s")

    @functools.partial(
        pl.kernel,
        out_type=jax.ShapeDtypeStruct((2, NUM_CORES, n, 16), jnp.float32),
        mesh=mesh,
        scratch_types=[
            pltpu.VMEM((nbw, b), jnp.int32),
            pltpu.VMEM((nbw, b), jnp.int32),
            pltpu.VMEM((b, 16), jnp.float32),
            pltpu.VMEM_SHARED((n, 16), jnp.float32),
            pltpu.VMEM_SHARED((n, 16), jnp.float32),
            pltpu.SemaphoreType.DMA,
        ],
        compiler_params=_SC_PARAMS,
    )
    def deg_kernel(src_hbm, dst_hbm, z_hbm, ones_hbm, out_hbm,
                   sidx, didx, ones_v, dsrc_sh, ddst_sh, sem):
        c = lax.axis_index("c")
        s = lax.axis_index("s")
        wid = c * NUM_SUBCORES + s
        r0 = s * zch
        pltpu.sync_copy(src_hbm.at[pl.ds(wid * nbw, nbw)], sidx)
        pltpu.sync_copy(dst_hbm.at[pl.ds(wid * nbw, nbw)], didx)
        pltpu.sync_copy(ones_hbm, ones_v)

        @pl.when(s < nz)
        def _():
            pltpu.sync_copy(z_hbm.at[pl.ds(r0, zch)],
                            dsrc_sh.at[pl.ds(r0, zch)])
            pltpu.sync_copy(z_hbm.at[pl.ds(r0, zch)],
                            ddst_sh.at[pl.ds(r0, zch)])

        plsc.subcore_barrier()

        @pl.loop(0, nbw)
        def _(i):
            pltpu.async_copy(ones_v, dsrc_sh.at[sidx.at[i]], sem, add=True)
            pltpu.async_copy(ones_v, ddst_sh.at[didx.at[i]], sem, add=True)

        @pl.loop(0, 2 * nbw)
        def _(i):
            pltpu.make_async_copy(ones_v, dsrc_sh.at[sidx.at[0]], sem).wait()

        plsc.subcore_barrier()

        @pl.when(s < nz)
        def _():
            pltpu.sync_copy(dsrc_sh.at[pl.ds(r0, zch)],
                            out_hbm.at[0, c, pl.ds(r0, zch)])
            pltpu.sync_copy(ddst_sh.at[pl.ds(r0, zch)],
                            out_hbm.at[1, c, pl.ds(r0, zch)])

    return deg_kernel(src2d, dst2d, zeros16, ones_rows)


def _sc_mega(u0, rio, src2d, dst2d, zeros, n, cp):
    nbt, b = src2d.shape
    nbw = nbt // NUM_WORKERS
    nz, zch = _row_chunks(n)
    mch = 200 if zch % 200 == 0 else zch
    ncnk = zch // mch
    mesh = plsc.VectorSubcoreMesh(core_axis_name="c", subcore_axis_name="s")
    pshape = jax.ShapeDtypeStruct((NUM_CORES, n, cp), jnp.float32)

    @functools.partial(
        pl.kernel,
        out_type=[pshape, pshape, pshape, pshape],
        mesh=mesh,
        scratch_types=[
            pltpu.VMEM((nbw, b), jnp.int32),
            pltpu.VMEM((nbw, b), jnp.int32),
            pltpu.VMEM((b, cp), jnp.float32),
            pltpu.VMEM((b, cp), jnp.float32),
            pltpu.VMEM((b, cp), jnp.float32),
            pltpu.VMEM((b, cp), jnp.float32),
            pltpu.VMEM((mch, cp), jnp.float32),
            pltpu.VMEM((mch, cp), jnp.float32),
            pltpu.VMEM((mch, cp), jnp.float32),
            pltpu.VMEM((mch, cp), jnp.float32),
            pltpu.VMEM((mch, cp), jnp.float32),
            pltpu.VMEM_SHARED((n, cp), jnp.float32),
            pltpu.SemaphoreType.DMA,
            pltpu.SemaphoreType.DMA,
            pltpu.SemaphoreType.DMA,
            pltpu.SemaphoreType.DMA,
            pltpu.SemaphoreType.DMA,
            pltpu.SemaphoreType.DMA,
            pltpu.SemaphoreType.DMA,
            pltpu.SemaphoreType.DMA,
            pltpu.SemaphoreType.DMA,
            pltpu.SemaphoreType.DMA,
            pltpu.SemaphoreType.REGULAR,
        ],
        compiler_params=_SC_PARAMS,
    )
    def mega_kernel(u0_hbm, rio_hbm, src_hbm, dst_hbm, z_hbm,
                    p1_hbm, p2_hbm, p3_hbm, u_scr,
                    sidx, didx, r0b, r1b, r2b, r3b,
                    i0, i1, i2, uca, ucb, acc_sh,
                    g0, g1, g2, g3, s0, s1, s2, s3, isem, osem, xsem):
        rows = (r0b, r1b, r2b, r3b)
        gsem = (g0, g1, g2, g3)
        ssem = (s0, s1, s2, s3)
        ucnks = (uca, ucb)
        c = lax.axis_index("c")
        s = lax.axis_index("s")
        wid = c * NUM_SUBCORES + s
        r0z = s * zch

        pltpu.sync_copy(src_hbm.at[pl.ds(wid * nbw, nbw)], sidx)
        pltpu.sync_copy(dst_hbm.at[pl.ds(wid * nbw, nbw)], didx)

        def zero_acc():
            @pl.when(s < nz)
            def _():
                pltpu.sync_copy(z_hbm.at[pl.ds(r0z, zch)],
                                acc_sh.at[pl.ds(r0z, zch)])

        def sparse_phase(u_ref):
            @pl.loop(0, nbw + 4, step=4)
            def _(v):
                for j in range(4):
                    i = v + j

                    @pl.when((i >= 4) & (i < nbw))
                    def _():
                        pltpu.make_async_copy(
                            rows[j], acc_sh.at[didx.at[0]], ssem[j]).wait()

                    @pl.when(i < nbw)
                    def _():
                        pltpu.async_copy(u_ref.at[sidx.at[i]], rows[j],
                                         gsem[j])

                    k2 = i - 2
                    jb = (j + 2) % 4

                    @pl.when((k2 >= 0) & (k2 < nbw))
                    def _():
                        pltpu.make_async_copy(
                            u_ref.at[sidx.at[0]], rows[jb], gsem[jb]).wait()
                        pltpu.async_copy(rows[jb], acc_sh.at[didx.at[k2]],
                                         ssem[jb], add=True)

            for j in range(4):
                pltpu.make_async_copy(rows[j], acc_sh.at[didx.at[0]],
                                      ssem[j]).wait()

        def writeback(p_hbm):
            @pl.when(s < nz)
            def _():
                pltpu.sync_copy(acc_sh.at[pl.ds(r0z, zch)],
                                p_hbm.at[c, pl.ds(r0z, zch)])

        def xsync():
            plsc.subcore_barrier()
            pl.semaphore_signal(xsem, 1, core_index=1 - c)
            pl.semaphore_wait(xsem, 1)

        def combine(p_ref):
            @pl.when(s < nz)
            def _():
                def fetch(t):
                    rr = r0z + t * mch
                    pltpu.async_copy(p_ref.at[0, pl.ds(rr, mch)], i0, isem)
                    pltpu.async_copy(p_ref.at[1, pl.ds(rr, mch)], i1, isem)
                    pltpu.async_copy(rio_hbm.at[pl.ds(rr, mch)], i2, isem)

                fetch(0)
                for t in range(ncnk):
                    rr = r0z + t * mch
                    uc = ucnks[t % 2]
                    for _ in range(3):
                        pltpu.make_async_copy(
                            rio_hbm.at[pl.ds(rr, mch)], i2, isem).wait()
                    if t >= 2:
                        pltpu.make_async_copy(
                            uc, u_scr.at[c, pl.ds(rr, mch)], osem).wait()

                    @pl.loop(0, mch, step=8)
                    def _(r):
                        for dr in range(8):
                            for q in range(cp // 16):
                                sl = (r + dr, pl.ds(q * 16, 16))
                                uc[sl] = i2[sl] * (i0[sl] + i1[sl])

                    if t + 1 < ncnk:
                        fetch(t + 1)
                    pltpu.async_copy(uc, u_scr.at[c, pl.ds(rr, mch)], osem)

                for _ in range(min(2, ncnk)):
                    pltpu.make_async_copy(
                        uca, u_scr.at[c, pl.ds(r0z, mch)], osem).wait()

        zero_acc()
        plsc.subcore_barrier()
        sparse_phase(u0_hbm)
        plsc.subcore_barrier()
        writeback(p1_hbm)
        zero_acc()
        xsync()

        combine(p1_hbm)
        plsc.subcore_barrier()
        sparse_phase(u_scr.at[c])
        plsc.subcore_barrier()
        writeback(p2_hbm)
        zero_acc()
        xsync()

        combine(p2_hbm)
        plsc.subcore_barrier()
        sparse_phase(u_scr.at[c])
        plsc.subcore_barrier()
        writeback(p3_hbm)

    res = mega_kernel(u0, rio, src2d, dst2d, zeros)
    return res[0], res[1], res[2]


def _tc_project(feature, w_padded, n, cp):
    def body(f_ref, w_ref, o_ref):
        o_ref[...] = jnp.dot(f_ref[...], w_ref[...],
                             preferred_element_type=jnp.float32)

    return pl.pallas_call(
        body, out_shape=jax.ShapeDtypeStruct((n, cp), jnp.float32),
    )(feature, w_padded)


def _tc_prep(y0, d_out0, d_out1, d_in0, d_in1, n, cp):
    def body(y_ref, do0, do1, di0, di1, u_ref, rio_ref, rin_ref):
        deg_out = do0[...] + do1[...]
        deg_in = di0[...] + di1[...]
        rout = lax.rsqrt(jnp.maximum(deg_out, 1.0))
        rin = lax.rsqrt(jnp.maximum(deg_in, 1.0))
        u_ref[...] = y_ref[...] * rout
        rio_ref[...] = jnp.broadcast_to(rin * rout, (n, cp))
        rin_ref[...] = rin

    return pl.pallas_call(
        body,
        out_shape=[
            jax.ShapeDtypeStruct((n, cp), jnp.float32),
            jax.ShapeDtypeStruct((n, cp), jnp.float32),
            jax.ShapeDtypeStruct((n, 1), jnp.float32),
        ],
    )(y0, d_out0, d_out1, d_in0, d_in1)


def _tc_final(p1, p2, p3, y0, rin, b2d, n, c_out):
    def body(p1_ref, p2_ref, p3_ref, y_ref, rin_ref, b_ref, o_ref):
        psum = ((p1_ref[0] + p1_ref[1]) + (p2_ref[0] + p2_ref[1])
                + (p3_ref[0] + p3_ref[1]))
        res = (y_ref[...] + psum * rin_ref[...]) * 0.25
        o_ref[...] = res[:, :c_out] + b_ref[...]

    return pl.pallas_call(
        body, out_shape=jax.ShapeDtypeStruct((n, c_out), jnp.float32),
    )(p1, p2, p3, y0, rin, b2d)


def kernel(feature, edge_index, W, b):
    n, d = feature.shape
    c_out = W.shape[1]
    e = edge_index.shape[1]
    cp = ((c_out + 15) // 16) * 16

    epw = e // NUM_WORKERS
    batch = _pick_batch(epw)
    nbt = e // batch

    src2d = edge_index[0].reshape(nbt, batch)
    dst2d = edge_index[1].reshape(nbt, batch)
    w_padded = jnp.pad(W, ((0, 0), (0, cp - c_out)))
    zeros16 = jnp.zeros((n, 16), jnp.float32)
    ones_rows = jnp.ones((batch, 16), jnp.float32)
    zeros_cp = jnp.zeros((n, cp), jnp.float32)
    b2d = b.reshape(1, c_out)

    y0 = _tc_project(feature, w_padded, n, cp)
    degs = _sc_degree(src2d, dst2d, zeros16, ones_rows, n)
    d_out0 = degs[0, 0, :, 0:1]
    d_out1 = degs[0, 1, :, 0:1]
    d_in0 = degs[1, 0, :, 0:1]
    d_in1 = degs[1, 1, :, 0:1]

    u0, rio, rin = _tc_prep(y0, d_out0, d_out1, d_in0, d_in1, n, cp)
    p1, p2, p3 = _sc_mega(u0, rio, src2d, dst2d, zeros_cp, n, cp)
    return _tc_final(p1, p2, p3, y0, rin, b2d, n, c_out)
